# Initial kernel scaffold; baseline (speedup 1.0000x reference)
import jax
import jax.numpy as jnp
from jax.experimental import pallas as pl


def kernel(contact_points, positions, rotations, scales):
    raise NotImplementedError("write your pallas kernel here")


# speedup vs baseline: 6.4101x; 6.4101x over previous
import functools

import jax
import jax.numpy as jnp
from jax import lax
from jax.experimental import pallas as pl
from jax.experimental.pallas import tpu as pltpu
from jax.experimental.pallas import tpu_sc as plsc

N = 2048
G = 65536
K = 8
CHUNK = 512
NCHUNK = G // CHUNK
BIG = 2 ** 30
MAX_DISTANCE = 0.05

SC_NC = 2
SC_NS = 16
SC_NW = SC_NC * SC_NS
B_TOTAL = N * K
B_PER_W = B_TOTAL // SC_NW
TD = 16


def _knn_body(cp_ref, pt_ref, cpb_ref, ptb_ref, ov_ref, oi_ref, cv_ref, ci_ref):
    i = pl.program_id(0)

    @pl.when(i == 0)
    def _init():
        cv_ref[...] = jnp.full((N, K), jnp.inf, jnp.float32)
        ci_ref[...] = jnp.full((N, K), BIG, jnp.int32)

    cp = cp_ref[...]
    q0 = cp[:, 0:1]
    q1 = cp[:, 1:2]
    q2 = cp[:, 2:3]
    p0 = pt_ref[0:1, :]
    p1 = pt_ref[1:2, :]
    p2 = pt_ref[2:3, :]
    qs = (q0 * q0 + q2 * q2) + q1 * q1
    ps = (p0 * p0 + p2 * p2) + p1 * p1
    cross = lax.dot_general(
        cpb_ref[...], ptb_ref[...], (((1,), (0,)), ((), ())),
        preferred_element_type=jnp.float32)
    blk = jnp.maximum((qs + ps) - 2.0 * cross, 0.0)

    gidx = lax.broadcasted_iota(jnp.int32, (1, CHUNK), 1) + i * CHUNK
    oc_v = cv_ref[...]
    oc_i = ci_ref[...]
    new_v = []
    new_i = []
    for _ in range(K):
        mb = jnp.min(blk, axis=1, keepdims=True)
        mc = jnp.min(oc_v, axis=1, keepdims=True)
        m = jnp.minimum(mb, mc)
        ibl = jnp.min(jnp.where(blk <= m, gidx, BIG), axis=1, keepdims=True)
        icr = jnp.min(jnp.where(oc_v <= m, oc_i, BIG), axis=1, keepdims=True)
        sel = jnp.minimum(ibl, icr)
        new_v.append(m)
        new_i.append(sel)
        blk = jnp.where(gidx == sel, jnp.inf, blk)
        oc_v = jnp.where(oc_i == sel, jnp.inf, oc_v)
    nv = jnp.concatenate(new_v, axis=1)
    ni = jnp.concatenate(new_i, axis=1)
    cv_ref[...] = nv
    ci_ref[...] = ni

    @pl.when(i == NCHUNK - 1)
    def _fin():
        ov_ref[...] = nv
        oi_ref[...] = ni


_knn_call = pl.pallas_call(
    _knn_body,
    grid=(NCHUNK,),
    in_specs=[
        pl.BlockSpec((N, 3), lambda i: (0, 0)),
        pl.BlockSpec((8, CHUNK), lambda i: (0, i)),
        pl.BlockSpec((N, 8), lambda i: (0, 0)),
        pl.BlockSpec((8, CHUNK), lambda i: (0, i)),
    ],
    out_specs=[
        pl.BlockSpec((N, K), lambda i: (0, 0)),
        pl.BlockSpec((N, K), lambda i: (0, 0)),
    ],
    out_shape=[
        jax.ShapeDtypeStruct((N, K), jnp.float32),
        jax.ShapeDtypeStruct((N, K), jnp.int32),
    ],
    scratch_shapes=[
        pltpu.VMEM((N, K), jnp.float32),
        pltpu.VMEM((N, K), jnp.int32),
    ],
    compiler_params=pltpu.CompilerParams(
        dimension_semantics=("arbitrary",),
    ),
)


def _gather_body(table_hbm, idx_hbm, out_hbm, idx_v, rows_v, sem):
    wid = lax.axis_index("s") * SC_NC + lax.axis_index("c")
    base = wid * B_PER_W
    pltpu.sync_copy(idx_hbm.at[pl.ds(base, B_PER_W)], idx_v)
    pltpu.async_copy(table_hbm.at[idx_v], rows_v, sem).wait()
    pltpu.sync_copy(rows_v, out_hbm.at[pl.ds(base, B_PER_W)])


def _make_sc_gather():
    return pl.kernel(
        _gather_body,
        mesh=plsc.VectorSubcoreMesh(core_axis_name="c", subcore_axis_name="s"),
        out_type=jax.ShapeDtypeStruct((B_TOTAL, TD), jnp.float32),
        scratch_types=[
            pltpu.VMEM((B_PER_W,), jnp.int32),
            pltpu.VMEM((B_PER_W, TD), jnp.float32),
            pltpu.SemaphoreType.DMA,
        ],
        compiler_params=pltpu.CompilerParams(use_tc_tiling_on_sc=False),
    )


def _mahal_body(g_ref, d2_ref, cp_ref, o_ref):
    px = g_ref[0]
    py = g_ref[1]
    pz = g_ref[2]
    rw = g_ref[3]
    rx = g_ref[4]
    ry = g_ref[5]
    rz = g_ref[6]
    sx = g_ref[7]
    sy = g_ref[8]
    sz = g_ref[9]
    d2w = d2_ref[...]
    dist = jnp.sqrt(jnp.maximum(d2w, 0.0))
    valid = dist < MAX_DISTANCE
    cx = cp_ref[0:1, :]
    cy = cp_ref[1:2, :]
    cz = cp_ref[2:3, :]
    nrm = jnp.sqrt(((rw * rw + rx * rx) + ry * ry) + rz * rz)
    inv = 1.0 / jnp.maximum(nrm, 1e-8)
    w = rw * inv
    x = rx * inv
    y = ry * inv
    z = rz * inv
    xx, yy, zz = x * x, y * y, z * z
    xy, xz, yz = x * y, x * z, y * z
    wx, wy, wz = w * x, w * y, w * z
    r00 = 1.0 - 2.0 * (yy + zz)
    r01 = 2.0 * (xy - wz)
    r02 = 2.0 * (xz + wy)
    r10 = 2.0 * (xy + wz)
    r11 = 1.0 - 2.0 * (xx + zz)
    r12 = 2.0 * (yz - wx)
    r20 = 2.0 * (xz - wy)
    r21 = 2.0 * (yz + wx)
    r22 = 1.0 - 2.0 * (xx + yy)

    def sig2(s):
        e = jnp.clip(jnp.exp(jnp.clip(s, -5.0, 5.0)), 1e-4, 1.0)
        return e * e + 1e-6

    d0 = sig2(sx)
    d1 = sig2(sy)
    d2 = sig2(sz)
    dx = cx - px
    dy = cy - py
    dz = cz - pz
    u0 = (r00 * dx + r10 * dy) + r20 * dz
    u1 = (r01 * dx + r11 * dy) + r21 * dz
    u2 = (r02 * dx + r12 * dy) + r22 * dz
    quad = (u0 * u0 / d0 + u1 * u1 / d1) + u2 * u2 / d2
    quad = jnp.minimum(quad, 1e6)
    mahal = jnp.sqrt(jnp.maximum(quad, 0.0))
    masked = jnp.where(valid, mahal, 1000.0)
    surf = jnp.min(masked, axis=0, keepdims=True)
    o_ref[...] = jnp.broadcast_to(surf, (K, N))


_mahal_call = pl.pallas_call(
    _mahal_body,
    in_specs=[
        pl.BlockSpec((TD, K, N), lambda: (0, 0, 0)),
        pl.BlockSpec((K, N), lambda: (0, 0)),
        pl.BlockSpec((8, N), lambda: (0, 0)),
    ],
    out_specs=pl.BlockSpec((K, N), lambda: (0, 0)),
    out_shape=jax.ShapeDtypeStruct((K, N), jnp.float32),
)


def kernel(contact_points, positions, rotations, scales):
    cp = contact_points
    pt = jnp.concatenate(
        [positions.T, jnp.zeros((5, G), jnp.float32)], axis=0)
    cpb = jnp.concatenate(
        [cp, jnp.zeros((N, 5), jnp.float32)], axis=1).astype(jnp.bfloat16)
    ptb = pt.astype(jnp.bfloat16)
    vals, idx = _knn_call(cp, pt, cpb, ptb)
    flat_idx = idx.reshape(B_TOTAL)
    table = jnp.concatenate(
        [positions, rotations, scales, jnp.zeros((G, 6), jnp.float32)],
        axis=1)
    gathered = _make_sc_gather()(table, flat_idx)
    comps = gathered.reshape(N, K, TD).transpose(2, 1, 0)
    d2t = vals.T
    cpt = jnp.concatenate(
        [cp.T, jnp.zeros((5, N), jnp.float32)], axis=0)
    out8 = _mahal_call(comps, d2t, cpt)
    return out8[0]


# speedup vs baseline: 6.8683x; 1.0715x over previous
import jax
import jax.numpy as jnp
from jax import lax
from jax.experimental import pallas as pl
from jax.experimental.pallas import tpu as pltpu
from jax.experimental.pallas import tpu_sc as plsc

N = 2048
G = 65536
K = 8
CHUNK = 512
NCHUNK = G // CHUNK
BIG = 2 ** 30
MAX_DISTANCE = 0.05

SC_NC = 2
SC_NS = 16
SC_NW = SC_NC * SC_NS
B_TOTAL = N * K
B_PER_W = B_TOTAL // SC_NW
TD = 16


def _merge_topk(blk, gidx, oc_v, oc_i):
    new_v = []
    new_i = []
    for _ in range(K):
        mb = jnp.min(blk, axis=1, keepdims=True)
        mc = jnp.min(oc_v, axis=1, keepdims=True)
        m = jnp.minimum(mb, mc)
        ibl = jnp.min(jnp.where(blk <= m, gidx, BIG), axis=1, keepdims=True)
        icr = jnp.min(jnp.where(oc_v <= m, oc_i, BIG), axis=1, keepdims=True)
        sel = jnp.minimum(ibl, icr)
        new_v.append(m)
        new_i.append(sel)
        blk = jnp.where(gidx == sel, jnp.inf, blk)
        oc_v = jnp.where(oc_i == sel, jnp.inf, oc_v)
    return jnp.concatenate(new_v, axis=1), jnp.concatenate(new_i, axis=1)


def _d2scan_body(cp_ref, pt_ref, cpb_ref, ptb_ref, d2_ref, cid_ref,
                 cv_ref, ci_ref):
    i = pl.program_id(0)

    @pl.when(i == 0)
    def _init():
        cv_ref[...] = jnp.full((N, K), jnp.inf, jnp.float32)
        ci_ref[...] = jnp.full((N, K), BIG, jnp.int32)

    cp = cp_ref[...]
    q0 = cp[:, 0:1]
    q1 = cp[:, 1:2]
    q2 = cp[:, 2:3]
    p0 = pt_ref[0:1, :]
    p1 = pt_ref[1:2, :]
    p2 = pt_ref[2:3, :]
    qs = (q0 * q0 + q2 * q2) + q1 * q1
    ps = (p0 * p0 + p2 * p2) + p1 * p1
    cross = lax.dot_general(
        cpb_ref[...], ptb_ref[...], (((1,), (0,)), ((), ())),
        preferred_element_type=jnp.float32)
    blk = jnp.maximum((qs + ps) - 2.0 * cross, 0.0)
    d2_ref[...] = blk

    m = jnp.min(blk, axis=1, keepdims=True)
    b_v = m
    b_i = jnp.full((N, 1), 0, jnp.int32) + i
    cv = cv_ref[...]
    ci = ci_ref[...]
    nvs = []
    nis = []
    for j in range(K):
        cj_v = cv[:, j:j + 1]
        cj_i = ci[:, j:j + 1]
        lt = (b_v < cj_v) | ((b_v == cj_v) & (b_i < cj_i))
        nvs.append(jnp.where(lt, b_v, cj_v))
        nis.append(jnp.where(lt, b_i, cj_i))
        b_v = jnp.where(lt, cj_v, b_v)
        b_i = jnp.where(lt, cj_i, b_i)
    cv_ref[...] = jnp.concatenate(nvs, axis=1)
    ci_ref[...] = jnp.concatenate(nis, axis=1)

    @pl.when(i == NCHUNK - 1)
    def _fin():
        cid_ref[...] = jnp.concatenate(nis, axis=1)


_d2scan_call = pl.pallas_call(
    _d2scan_body,
    grid=(NCHUNK,),
    in_specs=[
        pl.BlockSpec((N, 3), lambda i: (0, 0)),
        pl.BlockSpec((8, CHUNK), lambda i: (0, i)),
        pl.BlockSpec((N, 8), lambda i: (0, 0)),
        pl.BlockSpec((8, CHUNK), lambda i: (0, i)),
    ],
    out_specs=[
        pl.BlockSpec((N, CHUNK), lambda i: (0, i)),
        pl.BlockSpec((N, K), lambda i: (0, 0)),
    ],
    out_shape=[
        jax.ShapeDtypeStruct((N, G), jnp.float32),
        jax.ShapeDtypeStruct((N, K), jnp.int32),
    ],
    scratch_shapes=[
        pltpu.VMEM((N, K), jnp.float32),
        pltpu.VMEM((N, K), jnp.int32),
    ],
    compiler_params=pltpu.CompilerParams(
        dimension_semantics=("arbitrary",),
    ),
)


def _gather512_body(table_hbm, idx_hbm, out_hbm, idx_b, rows_v, sem):
    wid = lax.axis_index("s") * SC_NC + lax.axis_index("c")
    base = wid * B_PER_W
    for b in range(4):
        off = base + b * 128
        pltpu.sync_copy(idx_hbm.at[pl.ds(off, 128)], idx_b)
        pltpu.async_copy(table_hbm.at[idx_b], rows_v, sem).wait()
        pltpu.sync_copy(rows_v, out_hbm.at[pl.ds(off, 128)])


def _make_sc_gather512():
    return pl.kernel(
        _gather512_body,
        mesh=plsc.VectorSubcoreMesh(core_axis_name="c", subcore_axis_name="s"),
        out_type=jax.ShapeDtypeStruct((B_TOTAL, CHUNK), jnp.float32),
        scratch_types=[
            pltpu.VMEM((128,), jnp.int32),
            pltpu.VMEM((128, CHUNK), jnp.float32),
            pltpu.SemaphoreType.DMA,
        ],
        compiler_params=pltpu.CompilerParams(use_tc_tiling_on_sc=False),
    )


SEL_BLK = 2 * CHUNK
SEL_STEPS = (K * CHUNK) // SEL_BLK


def _sel_body(dg_ref, gx_ref, ov_ref, oi_ref, cv_ref, ci_ref):
    s = pl.program_id(0)

    @pl.when(s == 0)
    def _init():
        cv_ref[...] = jnp.full((N, K), jnp.inf, jnp.float32)
        ci_ref[...] = jnp.full((N, K), BIG, jnp.int32)

    gidx = gx_ref[...]
    blk = dg_ref[...]
    nv, ni = _merge_topk(blk, gidx, cv_ref[...], ci_ref[...])
    cv_ref[...] = nv
    ci_ref[...] = ni

    @pl.when(s == SEL_STEPS - 1)
    def _fin():
        ov_ref[...] = nv
        oi_ref[...] = ni


_sel_call = pl.pallas_call(
    _sel_body,
    grid=(SEL_STEPS,),
    in_specs=[
        pl.BlockSpec((N, SEL_BLK), lambda s: (0, s)),
        pl.BlockSpec((N, SEL_BLK), lambda s: (0, s)),
    ],
    out_specs=[
        pl.BlockSpec((N, K), lambda s: (0, 0)),
        pl.BlockSpec((N, K), lambda s: (0, 0)),
    ],
    out_shape=[
        jax.ShapeDtypeStruct((N, K), jnp.float32),
        jax.ShapeDtypeStruct((N, K), jnp.int32),
    ],
    scratch_shapes=[
        pltpu.VMEM((N, K), jnp.float32),
        pltpu.VMEM((N, K), jnp.int32),
    ],
    compiler_params=pltpu.CompilerParams(
        dimension_semantics=("arbitrary",),
    ),
)


def _gather_body(table_hbm, idx_hbm, out_hbm, idx_v, rows_v, sem):
    wid = lax.axis_index("s") * SC_NC + lax.axis_index("c")
    base = wid * B_PER_W
    pltpu.sync_copy(idx_hbm.at[pl.ds(base, B_PER_W)], idx_v)
    pltpu.async_copy(table_hbm.at[idx_v], rows_v, sem).wait()
    pltpu.sync_copy(rows_v, out_hbm.at[pl.ds(base, B_PER_W)])


def _make_sc_gather():
    return pl.kernel(
        _gather_body,
        mesh=plsc.VectorSubcoreMesh(core_axis_name="c", subcore_axis_name="s"),
        out_type=jax.ShapeDtypeStruct((B_TOTAL, TD), jnp.float32),
        scratch_types=[
            pltpu.VMEM((B_PER_W,), jnp.int32),
            pltpu.VMEM((B_PER_W, TD), jnp.float32),
            pltpu.SemaphoreType.DMA,
        ],
        compiler_params=pltpu.CompilerParams(use_tc_tiling_on_sc=False),
    )


def _mahal_body(g_ref, d2_ref, cp_ref, o_ref):
    px = g_ref[0]
    py = g_ref[1]
    pz = g_ref[2]
    rw = g_ref[3]
    rx = g_ref[4]
    ry = g_ref[5]
    rz = g_ref[6]
    sx = g_ref[7]
    sy = g_ref[8]
    sz = g_ref[9]
    d2w = d2_ref[...]
    dist = jnp.sqrt(jnp.maximum(d2w, 0.0))
    valid = dist < MAX_DISTANCE
    cx = cp_ref[0:1, :]
    cy = cp_ref[1:2, :]
    cz = cp_ref[2:3, :]
    nrm = jnp.sqrt(((rw * rw + rx * rx) + ry * ry) + rz * rz)
    inv = 1.0 / jnp.maximum(nrm, 1e-8)
    w = rw * inv
    x = rx * inv
    y = ry * inv
    z = rz * inv
    xx, yy, zz = x * x, y * y, z * z
    xy, xz, yz = x * y, x * z, y * z
    wx, wy, wz = w * x, w * y, w * z
    r00 = 1.0 - 2.0 * (yy + zz)
    r01 = 2.0 * (xy - wz)
    r02 = 2.0 * (xz + wy)
    r10 = 2.0 * (xy + wz)
    r11 = 1.0 - 2.0 * (xx + zz)
    r12 = 2.0 * (yz - wx)
    r20 = 2.0 * (xz - wy)
    r21 = 2.0 * (yz + wx)
    r22 = 1.0 - 2.0 * (xx + yy)

    def sig2(s):
        e = jnp.clip(jnp.exp(jnp.clip(s, -5.0, 5.0)), 1e-4, 1.0)
        return e * e + 1e-6

    d0 = sig2(sx)
    d1 = sig2(sy)
    d2 = sig2(sz)
    dx = cx - px
    dy = cy - py
    dz = cz - pz
    u0 = (r00 * dx + r10 * dy) + r20 * dz
    u1 = (r01 * dx + r11 * dy) + r21 * dz
    u2 = (r02 * dx + r12 * dy) + r22 * dz
    quad = (u0 * u0 / d0 + u1 * u1 / d1) + u2 * u2 / d2
    quad = jnp.minimum(quad, 1e6)
    mahal = jnp.sqrt(jnp.maximum(quad, 0.0))
    masked = jnp.where(valid, mahal, 1000.0)
    surf = jnp.min(masked, axis=0, keepdims=True)
    o_ref[...] = jnp.broadcast_to(surf, (K, N))


_mahal_call = pl.pallas_call(
    _mahal_body,
    in_specs=[
        pl.BlockSpec((TD, K, N), lambda: (0, 0, 0)),
        pl.BlockSpec((K, N), lambda: (0, 0)),
        pl.BlockSpec((8, N), lambda: (0, 0)),
    ],
    out_specs=pl.BlockSpec((K, N), lambda: (0, 0)),
    out_shape=jax.ShapeDtypeStruct((K, N), jnp.float32),
)


def kernel(contact_points, positions, rotations, scales):
    cp = contact_points
    pt = jnp.concatenate(
        [positions.T, jnp.zeros((5, G), jnp.float32)], axis=0)
    cpb = jnp.concatenate(
        [cp, jnp.zeros((N, 5), jnp.float32)], axis=1).astype(jnp.bfloat16)
    ptb = pt.astype(jnp.bfloat16)
    d2full, cid = _d2scan_call(cp, pt, cpb, ptb)

    rowidx = (jnp.arange(N, dtype=jnp.int32)[:, None] * NCHUNK
              + cid).reshape(B_TOTAL)
    dtab = d2full.reshape(N * NCHUNK, CHUNK)
    dg = _make_sc_gather512()(dtab, rowidx)
    dgw = dg.reshape(N, K * CHUNK)

    gx = (cid[:, :, None] * CHUNK
          + jnp.arange(CHUNK, dtype=jnp.int32)[None, None, :]
          ).reshape(N, K * CHUNK)
    vals, idx = _sel_call(dgw, gx)

    flat_idx = idx.reshape(B_TOTAL)
    table = jnp.concatenate(
        [positions, rotations, scales, jnp.zeros((G, 6), jnp.float32)],
        axis=1)
    gathered = _make_sc_gather()(table, flat_idx)
    comps = gathered.reshape(N, K, TD).transpose(2, 1, 0)
    d2t = vals.T
    cpt = jnp.concatenate(
        [cp.T, jnp.zeros((5, N), jnp.float32)], axis=0)
    out8 = _mahal_call(comps, d2t, cpt)
    return out8[0]
